# baseline (speedup 1.0000x reference)
import functools

import jax
import jax.numpy as jnp
from jax import lax
from jax.experimental import pallas as pl
from jax.experimental.pallas import tpu as pltpu
from jax.experimental.pallas import tpu_sc as plsc

_F = 26
_V = 100000
_E = 64
_B = 4096
_OUTW = _F * _E
_ROWS = _B * _F
_TRASH = _ROWS
_SROWS = _ROWS + 16

_NC = 2
_NS = 16
_NW = _NC * _NS
_NWIN = (_V + 127) // 128
_WPW = 25
_LASTWIN = _NWIN - 1
_NSS = 8

_I16 = lambda: lax.iota(jnp.int32, 16)


def _shift_right1(v16, fill):
    iota = _I16()
    idx = jnp.clip(iota - 1, 0, 15)
    sh = lax.gather(
        v16, idx[:, None],
        dimension_numbers=lax.GatherDimensionNumbers(
            offset_dims=(), collapsed_slice_dims=(0,), start_index_map=(0,)),
        slice_sizes=(1,),
        mode=lax.GatherScatterMode.PROMISE_IN_BOUNDS)
    return jnp.where(iota >= 1, sh, fill)


def _rank_equal_prefix(c16):
    iota = _I16()
    r = jnp.zeros((16,), jnp.int32)
    for k in range(1, 16):
        idx = jnp.clip(iota - k, 0, 15)
        sh = lax.gather(
            c16, idx[:, None],
            dimension_numbers=lax.GatherDimensionNumbers(
                offset_dims=(), collapsed_slice_dims=(0,),
                start_index_map=(0,)),
            slice_sizes=(1,),
            mode=lax.GatherScatterMode.PROMISE_IN_BOUNDS)
        r = r + jnp.where((iota >= k) & (sh == c16), 1, 0).astype(jnp.int32)
    return r


def _body_a(table, xt, tail, outf, xv, cnt, offs_base, offs_run, mvs, mbs, bb,
            mbuf, pstage, semb0, semb1, *semss):
    wid = lax.axis_index("s") * _NC + lax.axis_index("c")
    c0 = wid * _WPW
    nwin = jnp.minimum(_WPW, _NWIN - c0)
    iota = _I16()
    ones = jnp.ones((16,), jnp.int32)
    zeros16 = jnp.zeros((16,), jnp.int32)
    trash16 = jnp.full((16,), _TRASH, jnp.int32) + iota

    def batch_fire(slot):
        @pl.when(slot == 0)
        def _():
            pltpu.async_copy(mbuf.at[0], outf.at[pstage.at[0]], semss[0])

        @pl.when(slot != 0)
        def _():
            pltpu.async_copy(mbuf.at[1], outf.at[pstage.at[1]], semss[1])

    def batch_wait(slot):
        @pl.when(slot == 0)
        def _():
            pltpu.make_async_copy(
                mbuf.at[0], outf.at[pstage.at[0]], semss[0]).wait()

        @pl.when(slot != 0)
        def _():
            pltpu.make_async_copy(
                mbuf.at[1], outf.at[pstage.at[1]], semss[1]).wait()

    for sl in range(2):
        for q in range(_NSS):
            pstage[sl, pl.ds(q * 16, 16)] = trash16

    def fire_block(f, c, par):
        lane0 = pl.multiple_of((c0 + c) * 128, 128)
        row0 = pl.multiple_of(f * _E, 8)

        @pl.when(c0 + c < _LASTWIN)
        def _():
            @pl.when(par == 0)
            def _():
                pltpu.async_copy(
                    table.at[pl.ds(row0, _E), pl.ds(lane0, 128)],
                    bb.at[0], semb0)

            @pl.when(par != 0)
            def _():
                pltpu.async_copy(
                    table.at[pl.ds(row0, _E), pl.ds(lane0, 128)],
                    bb.at[1], semb1)

        @pl.when(c0 + c == _LASTWIN)
        def _():
            @pl.when(par == 0)
            def _():
                pltpu.async_copy(
                    tail.at[pl.ds(row0, _E), pl.ds(0, 128)], bb.at[0], semb0)

            @pl.when(par != 0)
            def _():
                pltpu.async_copy(
                    tail.at[pl.ds(row0, _E), pl.ds(0, 128)], bb.at[1], semb1)

    def wait_block(f, c, par):
        lane0 = pl.multiple_of((c0 + c) * 128, 128)
        row0 = pl.multiple_of(f * _E, 8)

        @pl.when(c0 + c < _LASTWIN)
        def _():
            @pl.when(par == 0)
            def _():
                pltpu.make_async_copy(
                    table.at[pl.ds(row0, _E), pl.ds(lane0, 128)],
                    bb.at[0], semb0).wait()

            @pl.when(par != 0)
            def _():
                pltpu.make_async_copy(
                    table.at[pl.ds(row0, _E), pl.ds(lane0, 128)],
                    bb.at[1], semb1).wait()

        @pl.when(c0 + c == _LASTWIN)
        def _():
            @pl.when(par == 0)
            def _():
                pltpu.make_async_copy(
                    tail.at[pl.ds(row0, _E), pl.ds(0, 128)],
                    bb.at[0], semb0).wait()

            @pl.when(par != 0)
            def _():
                pltpu.make_async_copy(
                    tail.at[pl.ds(row0, _E), pl.ds(0, 128)],
                    bb.at[1], semb1).wait()

    def field_body(f, spar):
        pltpu.sync_copy(xt.at[f], xv)
        fire_block(f, 0, 0)

        cnt[pl.ds(0, 16)] = zeros16
        cnt[pl.ds(16, 16)] = zeros16

        def h_body(g, carry):
            v16 = xv[pl.ds(g * 16, 16)]
            cw = (v16 >> 7) - c0
            msk = (cw >= 0) & (cw < nwin)
            cwc = jnp.clip(cw, 0, 31)
            plsc.addupdate_scatter(cnt, [cwc], ones, mask=msk)
            return carry

        lax.fori_loop(0, _B // 16, h_body, 0)

        inc0 = plsc.cumsum(cnt[pl.ds(0, 16)])
        tot0 = inc0[15]
        inc1 = plsc.cumsum(cnt[pl.ds(16, 16)]) + tot0
        excl0 = _shift_right1(inc0, 0)
        excl1 = _shift_right1(inc1, tot0)
        offs_base[pl.ds(0, 16)] = excl0
        offs_base[pl.ds(16, 16)] = excl1
        offs_base[pl.ds(32, 16)] = jnp.full((16,), inc1[15], jnp.int32)
        offs_run[pl.ds(0, 16)] = excl0
        offs_run[pl.ds(16, 16)] = excl1
        nm = inc1[15]

        def p_body(g, carry):
            v16 = xv[pl.ds(g * 16, 16)]
            b16 = g * 16 + iota
            cw = (v16 >> 7) - c0
            msk = (cw >= 0) & (cw < nwin)
            cwc = jnp.clip(cw, 0, 31)
            pc = plsc.all_reduce_population_count(msk)[0]

            @pl.when(pc > 0)
            def _():
                rank = lax.cond(
                    pc > 1,
                    lambda: _rank_equal_prefix(jnp.where(msk, cwc, -1 - iota)),
                    lambda: zeros16,
                )
                slot = plsc.load_gather(offs_run, [cwc]) + rank
                slot = jnp.clip(slot, 0, _B - 1)
                plsc.addupdate_scatter(offs_run, [cwc], ones, mask=msk)
                plsc.store_scatter(mvs, [slot], v16, mask=msk)
                plsc.store_scatter(mbs, [slot], b16, mask=msk)

            return carry

        lax.fori_loop(0, _B // 16, p_body, 0)

        def block_body(c, spar2):
            par = c & 1

            @pl.when(c + 1 < nwin)
            def _():
                fire_block(f, c + 1, 1 - par)

            wait_block(f, c, par)
            lo = offs_base[pl.ds(c, 16)][0]
            hi = offs_base[pl.ds(c + 1, 16)][0]
            kstart = lo >> 4
            kend = lax.select(hi > lo, (hi + 15) >> 4, kstart)

            def group_body(k, sp):
                pos = k * 16 + iota
                mskg = (pos >= lo) & (pos < hi)
                v16 = mvs[pl.ds(k * 16, 16)]
                b16 = mbs[pl.ds(k * 16, 16)]
                lane = v16 & 127
                par16 = jnp.full((16,), par, jnp.int32)
                slot = (sp >> 3) & 1
                sub = sp & 7

                @pl.when((sub == 0) & (sp >= 16))
                def _():
                    batch_wait(slot)

                row16 = sub * 16 + iota
                slot16 = jnp.full((16,), slot, jnp.int32)
                for e in range(_E):
                    e16 = jnp.full((16,), e, jnp.int32)
                    g16 = plsc.load_gather(bb, [par16, e16, lane])
                    plsc.store_scatter(mbuf, [slot16, row16, e16], g16)

                p16 = jnp.where(mskg, b16 * _F + f, trash16)
                st = pl.multiple_of(sub * 16, 16)

                @pl.when(slot == 0)
                def _():
                    pstage[0, pl.ds(st, 16)] = p16

                @pl.when(slot != 0)
                def _():
                    pstage[1, pl.ds(st, 16)] = p16

                @pl.when(sub == 7)
                def _():
                    batch_fire(slot)

                return sp + 1

            return lax.fori_loop(kstart, kend, group_body, spar2)

        return lax.fori_loop(0, nwin, block_body, spar)

    spar_final = lax.fori_loop(0, _F, field_body, 0)

    def pad_body(d, sp):
        slot = (sp >> 3) & 1
        sub = sp & 7
        st = pl.multiple_of(sub * 16, 16)

        @pl.when(slot == 0)
        def _():
            pstage[0, pl.ds(st, 16)] = trash16

        @pl.when(slot != 0)
        def _():
            pstage[1, pl.ds(st, 16)] = trash16

        @pl.when(sub == 7)
        def _():
            batch_fire(slot)

        return sp + 1

    ndum = (8 - (spar_final & 7)) & 7
    lax.fori_loop(0, ndum, pad_body, spar_final)
    batch_wait(0)
    batch_wait(1)


_BCH = 208
_BNCH = _ROWS // _NW // _BCH


def _body_b(outf, out3, gbuf, obuf, semg0, semg1, semw0, semw1):
    wid = lax.axis_index("s") * _NC + lax.axis_index("c")
    base = wid * _BCH * _BNCH
    obase = wid * (_B // _NW)

    def fire_g(k, par):
        r0 = pl.multiple_of(base + k * _BCH, 8)

        @pl.when(par == 0)
        def _():
            pltpu.async_copy(outf.at[pl.ds(r0, _BCH)], gbuf.at[0], semg0)

        @pl.when(par != 0)
        def _():
            pltpu.async_copy(outf.at[pl.ds(r0, _BCH)], gbuf.at[1], semg1)

    def wait_g(k, par):
        r0 = pl.multiple_of(base + k * _BCH, 8)

        @pl.when(par == 0)
        def _():
            pltpu.make_async_copy(
                outf.at[pl.ds(r0, _BCH)], gbuf.at[0], semg0).wait()

        @pl.when(par != 0)
        def _():
            pltpu.make_async_copy(
                outf.at[pl.ds(r0, _BCH)], gbuf.at[1], semg1).wait()

    def fire_w(k, par):
        o0 = obase + k * 8

        @pl.when(par == 0)
        def _():
            pltpu.async_copy(obuf.at[0], out3.at[pl.ds(o0, 8), 0], semw0)

        @pl.when(par != 0)
        def _():
            pltpu.async_copy(obuf.at[1], out3.at[pl.ds(o0, 8), 0], semw1)

    def wait_w(k, par):
        o0 = obase + k * 8

        @pl.when(par == 0)
        def _():
            pltpu.make_async_copy(
                obuf.at[0], out3.at[pl.ds(o0, 8), 0], semw0).wait()

        @pl.when(par != 0)
        def _():
            pltpu.make_async_copy(
                obuf.at[1], out3.at[pl.ds(o0, 8), 0], semw1).wait()

    fire_g(0, 0)

    def chunk_body(k, carry):
        par = k & 1

        @pl.when(k + 1 < _BNCH)
        def _():
            fire_g(k + 1, 1 - par)

        wait_g(k, par)

        @pl.when(k >= 2)
        def _():
            wait_w(k - 2, par)

        def row_body(j, carry2):
            jr = j // _F
            jc = (j - jr * _F) * _E

            @pl.when(par == 0)
            def _():
                for k2 in range(_E // 16):
                    obuf[0, jr, pl.ds(jc + k2 * 16, 16)] = gbuf[
                        0, j, pl.ds(k2 * 16, 16)]

            @pl.when(par != 0)
            def _():
                for k2 in range(_E // 16):
                    obuf[1, jr, pl.ds(jc + k2 * 16, 16)] = gbuf[
                        1, j, pl.ds(k2 * 16, 16)]

            return carry2

        lax.fori_loop(0, _BCH, row_body, 0)
        fire_w(k, par)
        return carry

    lax.fori_loop(0, _BNCH, chunk_body, 0)
    wait_w(_BNCH - 2, _BNCH & 1)
    wait_w(_BNCH - 1, 1 - (_BNCH & 1))


@jax.jit
def _sc_embed(table_t, xt, tail):
    mesh = plsc.VectorSubcoreMesh(core_axis_name="c", subcore_axis_name="s")
    cp = pltpu.CompilerParams(
        use_tc_tiling_on_sc=True, needs_layout_passes=False)

    fa = functools.partial(
        pl.kernel,
        mesh=mesh,
        out_type=jax.ShapeDtypeStruct((_SROWS, 128), jnp.float32),
        scratch_types=[
            pltpu.VMEM((_B,), jnp.int32),
            pltpu.VMEM((32,), jnp.int32),
            pltpu.VMEM((48,), jnp.int32),
            pltpu.VMEM((32,), jnp.int32),
            pltpu.VMEM((_B,), jnp.int32),
            pltpu.VMEM((_B,), jnp.int32),
            pltpu.VMEM((2, _E, 128), jnp.float32),
            pltpu.VMEM((2, 8 * 16, 128), jnp.float32),
            pltpu.VMEM((2, 8 * 16), jnp.int32),
            pltpu.SemaphoreType.DMA,
            pltpu.SemaphoreType.DMA,
        ]
        + [pltpu.SemaphoreType.DMA] * 2,
        compiler_params=cp,
    )(_body_a)
    outf = fa(table_t, xt, tail)

    fb = functools.partial(
        pl.kernel,
        mesh=mesh,
        out_type=jax.ShapeDtypeStruct((_B, 1, _OUTW), jnp.float32),
        scratch_types=[
            pltpu.VMEM((2, _BCH, 128), jnp.float32),
            pltpu.VMEM((2, 8, _OUTW), jnp.float32),
            pltpu.SemaphoreType.DMA,
            pltpu.SemaphoreType.DMA,
            pltpu.SemaphoreType.DMA,
            pltpu.SemaphoreType.DMA,
        ],
        compiler_params=cp,
    )(_body_b)
    return fb(outf)


def kernel(X, tables):
    table_t = tables.transpose(0, 2, 1).reshape(_F * _E, _V)
    xt = X.T
    tail = jnp.pad(
        lax.slice(table_t, (0, _V - 32), (_F * _E, _V)), ((0, 0), (0, 96)))
    return _sc_embed(table_t, xt, tail)


# speedup vs baseline: 1.0783x; 1.0783x over previous
import functools

import jax
import jax.numpy as jnp
from jax import lax
from jax.experimental import pallas as pl
from jax.experimental.pallas import tpu as pltpu
from jax.experimental.pallas import tpu_sc as plsc

_NUM_FIELDS = 26
_VOCAB = 100000
_EMB = 64
_BATCH = 4096

_NC = 2
_NS = 16
_NW = _NC * _NS
_ROWS = _BATCH * _NUM_FIELDS
_RPW = _ROWS // _NW
_CHUNK = 104
_NCHUNK = _RPW // _CHUNK
_NBUF = 8
_NROUND = _NCHUNK // _NBUF
_LANES = 16


def _body(table, xh, offh, out, xv, offv, rows, *sems):
    wid = lax.axis_index("s") * _NC + lax.axis_index("c")
    base = wid * _RPW

    pltpu.sync_copy(xh.at[pl.ds(base, _RPW)], xv)
    pltpu.sync_copy(offh, offv)

    def add_body(i, carry):
        sl = pl.ds(i * _LANES, _LANES)
        xv[sl] = xv[sl] + offv[sl]
        return carry

    lax.fori_loop(0, _RPW // _LANES, add_body, 0)

    def chunk_off(g):
        return pl.multiple_of(g * _CHUNK, 8)

    def fire_gather(g, b):
        pltpu.async_copy(
            table.at[xv.at[pl.ds(chunk_off(g), _CHUNK)]], rows.at[b], sems[b]
        )

    def wait_gather(g, b):
        pltpu.make_async_copy(
            table.at[xv.at[pl.ds(chunk_off(g), _CHUNK)]], rows.at[b], sems[b]
        ).wait()

    def fire_wb(g, b):
        pltpu.async_copy(
            rows.at[b], out.at[pl.ds(base + chunk_off(g), _CHUNK)], sems[b]
        )

    def wait_wb(g, b):
        pltpu.make_async_copy(
            rows.at[b], out.at[pl.ds(base + chunk_off(g), _CHUNK)], sems[b]
        ).wait()

    for b in range(_NBUF):
        fire_gather(b, b)

    def round_body(t, carry):
        for b in range(_NBUF):
            g = t * _NBUF + b
            wait_gather(g, b)
            fire_wb(g, b)
            wait_wb(g, b)
            fire_gather(g + _NBUF, b)
        return carry

    lax.fori_loop(0, _NROUND - 1, round_body, 0)

    for b in range(_NBUF):
        g = (_NROUND - 1) * _NBUF + b
        wait_gather(g, b)
        fire_wb(g, b)
        wait_wb(g, b)


@jax.jit
def _sc_gather(table_flat, x_flat, offs):
    mesh = plsc.VectorSubcoreMesh(core_axis_name="c", subcore_axis_name="s")
    f = functools.partial(
        pl.kernel,
        mesh=mesh,
        out_type=jax.ShapeDtypeStruct((_ROWS, _EMB), jnp.float32),
        scratch_types=[
            pltpu.VMEM((_RPW,), jnp.int32),
            pltpu.VMEM((_RPW,), jnp.int32),
            pltpu.VMEM((_NBUF, _CHUNK, _EMB), jnp.float32),
        ]
        + [pltpu.SemaphoreType.DMA] * _NBUF,
        compiler_params=pltpu.CompilerParams(use_tc_tiling_on_sc=False),
    )(_body)
    return f(table_flat, x_flat, offs)


def kernel(X, tables):
    table_flat = tables.reshape(_NUM_FIELDS * _VOCAB, _EMB)
    x_flat = X.reshape(-1)
    offs = jnp.tile(
        jnp.arange(_NUM_FIELDS, dtype=jnp.int32) * _VOCAB, _RPW // _NUM_FIELDS
    )
    out = _sc_gather(table_flat, x_flat, offs)
    return out.reshape(_BATCH, 1, _NUM_FIELDS * _EMB)
